# baseline (speedup 1.0000x reference)
import functools

import jax
import jax.numpy as jnp
from jax import lax
from jax.experimental import pallas as pl
from jax.experimental.pallas import tpu as pltpu
from jax.experimental.pallas import tpu_sc as plsc

D = 128
C = 80
L = 16


def _sc_scores(x, src, dst):
    E = src.shape[0]
    N = x.shape[0]
    info = plsc.get_sparse_core_info()
    NS = info.num_subcores
    NW = info.num_cores * NS
    per_w = E // NW
    n_chunks = per_w // C
    rows_per_s = (N // NS) // 8 * 8
    rows_tail = N - rows_per_s * NS
    mesh = plsc.VectorSubcoreMesh(core_axis_name="c", subcore_axis_name="s")

    @functools.partial(
        pl.kernel,
        mesh=mesh,
        compiler_params=pltpu.CompilerParams(needs_layout_passes=False),
        out_type=jax.ShapeDtypeStruct((E,), jnp.float32),
        scratch_types=[
            pltpu.VMEM_SHARED((N, D), jnp.float32),
            pltpu.VMEM((2, C), jnp.int32),
            pltpu.VMEM((2, C), jnp.int32),
            pltpu.VMEM((C, D), jnp.float32),
            pltpu.VMEM((C, D), jnp.float32),
            pltpu.VMEM((C, D), jnp.float32),
            pltpu.VMEM((C, D), jnp.float32),
            pltpu.VMEM((2, C), jnp.float32),
            pltpu.SemaphoreType.DMA,
            pltpu.SemaphoreType.DMA,
            pltpu.SemaphoreType.DMA,
            pltpu.SemaphoreType.DMA,
            pltpu.SemaphoreType.DMA,
            pltpu.SemaphoreType.DMA,
        ],
    )
    def k(x_hbm, src_hbm, dst_hbm, out_hbm,
          x_sh, sidx, didx, sr0, dr0, sr1, dr1, outv,
          rsem0, rsem1, isem0, isem1, osem0, osem1):
        sid = lax.axis_index("s")
        wid = sid * info.num_cores + lax.axis_index("c")
        base = wid * per_w
        lane = lax.broadcasted_iota(jnp.int32, (L,), 0)
        rbufs = ((sr0, dr0, rsem0), (sr1, dr1, rsem1))
        isems = (isem0, isem1)
        osems = (osem0, osem1)

        rbase = sid * rows_per_s
        pltpu.sync_copy(x_hbm.at[pl.ds(rbase, rows_per_s)],
                        x_sh.at[pl.ds(rbase, rows_per_s)])
        if rows_tail:
            @pl.when(sid == 0)
            def _():
                pltpu.sync_copy(x_hbm.at[pl.ds(rows_per_s * NS, rows_tail)],
                                x_sh.at[pl.ds(rows_per_s * NS, rows_tail)])
        plsc.subcore_barrier()

        def fire_idx(ci, b):
            pltpu.async_copy(src_hbm.at[pl.ds(base + ci * C, C)],
                             sidx.at[b], isems[b])
            pltpu.async_copy(dst_hbm.at[pl.ds(base + ci * C, C)],
                             didx.at[b], isems[b])

        def wait_idx(b):
            pltpu.make_async_copy(src_hbm.at[pl.ds(0, C)], sidx.at[b],
                                  isems[b]).wait()
            pltpu.make_async_copy(dst_hbm.at[pl.ds(0, C)], didx.at[b],
                                  isems[b]).wait()

        def fire_rows(b):
            sr, dr, sem = rbufs[b]
            pltpu.async_copy(x_sh.at[sidx.at[b]], sr, sem)
            pltpu.async_copy(x_sh.at[didx.at[b]], dr, sem)

        def wait_rows(b):
            sr, dr, sem = rbufs[b]
            pltpu.make_async_copy(x_sh.at[sidx.at[b]], sr, sem).wait()
            pltpu.make_async_copy(x_sh.at[didx.at[b]], dr, sem).wait()

        def compute(ci, b):
            sr, dr, _ = rbufs[b]
            for g in range(C // L):
                row = g * L + lane

                def f_body(f, acc):
                    col = (f + lane) & (D - 1)
                    s = plsc.load_gather(sr, [row, col])
                    d_ = plsc.load_gather(dr, [row, col])
                    return acc + s * d_

                acc = lax.fori_loop(0, D, f_body, jnp.zeros((L,), jnp.float32),
                                    unroll=4)
                outv[b, pl.ds(g * L, L)] = acc
            pltpu.async_copy(outv.at[b], out_hbm.at[pl.ds(base + ci * C, C)],
                             osems[b])

        def wait_out(b):
            pltpu.make_async_copy(outv.at[b], out_hbm.at[pl.ds(0, C)],
                                  osems[b]).wait()

        fire_idx(0, 0)
        fire_idx(1, 1)
        wait_idx(0)
        fire_rows(0)

        def body(ci, carry):
            b = lax.rem(ci, 2)

            def even(_):
                wait_rows(0)

                @pl.when(ci < n_chunks - 2)
                def _():
                    fire_idx(ci + 2, 0)

                @pl.when(ci < n_chunks - 1)
                def _():
                    wait_idx(1)
                    fire_rows(1)

                @pl.when(ci >= 2)
                def _():
                    wait_out(0)
                compute(ci, 0)
                return 0

            def odd(_):
                wait_rows(1)

                @pl.when(ci < n_chunks - 2)
                def _():
                    fire_idx(ci + 2, 1)

                @pl.when(ci < n_chunks - 1)
                def _():
                    wait_idx(0)
                    fire_rows(0)

                @pl.when(ci >= 2)
                def _():
                    wait_out(1)
                compute(ci, 1)
                return 0

            lax.cond(b == 0, even, odd, 0)
            return carry

        lax.fori_loop(0, n_chunks, body, 0)
        wait_out(0)
        wait_out(1)

    return k(x, src, dst)


def kernel(x, edge_index):
    ei = edge_index.astype(jnp.int32)
    scores = _sc_scores(x, ei[0], ei[1])
    return scores.reshape(-1, 1)


# speedup vs baseline: 1.2594x; 1.2594x over previous
import functools

import jax
import jax.numpy as jnp
from jax import lax
from jax.experimental import pallas as pl
from jax.experimental.pallas import tpu as pltpu
from jax.experimental.pallas import tpu_sc as plsc

D = 128
C = 80
L = 16
NBUF = 4


def _sc_scores(x, src, dst):
    E = src.shape[0]
    info = plsc.get_sparse_core_info()
    NW = info.num_cores * info.num_subcores
    per_w = E // NW
    n_chunks = per_w // C
    n_super = (n_chunks - (NBUF - 1)) // NBUF
    mesh = plsc.VectorSubcoreMesh(core_axis_name="c", subcore_axis_name="s")

    @functools.partial(
        pl.kernel,
        mesh=mesh,
        compiler_params=pltpu.CompilerParams(needs_layout_passes=False),
        out_type=jax.ShapeDtypeStruct((E,), jnp.float32),
        scratch_types=[
            pltpu.VMEM((per_w,), jnp.int32),
            pltpu.VMEM((per_w,), jnp.int32),
            [pltpu.VMEM((C, D), jnp.float32) for _ in range(NBUF)],
            [pltpu.VMEM((C, D), jnp.float32) for _ in range(NBUF)],
            pltpu.VMEM((per_w,), jnp.float32),
            [pltpu.SemaphoreType.DMA for _ in range(NBUF)],
        ],
    )
    def k(x_hbm, src_hbm, dst_hbm, out_hbm,
          sidx, didx, srs, drs, outv, sems):
        wid = lax.axis_index("s") * info.num_cores + lax.axis_index("c")
        base = wid * per_w
        lane = lax.broadcasted_iota(jnp.int32, (L,), 0)

        pltpu.sync_copy(src_hbm.at[pl.ds(base, per_w)], sidx)
        pltpu.sync_copy(dst_hbm.at[pl.ds(base, per_w)], didx)

        def fire(ci, b):
            pltpu.async_copy(x_hbm.at[sidx.at[pl.ds(ci * C, C)]], srs[b], sems[b])
            pltpu.async_copy(x_hbm.at[didx.at[pl.ds(ci * C, C)]], drs[b], sems[b])

        def wait(b):
            pltpu.make_async_copy(x_hbm.at[sidx.at[pl.ds(0, C)]], srs[b],
                                  sems[b]).wait()
            pltpu.make_async_copy(x_hbm.at[didx.at[pl.ds(0, C)]], drs[b],
                                  sems[b]).wait()

        def compute(ci, b):
            sr, dr = srs[b], drs[b]
            for g in range(C // L):
                row = g * L + lane

                def f_body(f, acc):
                    col = (f + lane) & (D - 1)
                    s = plsc.load_gather(sr, [row, col])
                    d_ = plsc.load_gather(dr, [row, col])
                    return acc + s * d_

                acc = lax.fori_loop(0, D, f_body, jnp.zeros((L,), jnp.float32),
                                    unroll=4)
                outv[pl.ds(ci * C + g * L, L)] = acc

        for b in range(NBUF - 1):
            fire(b, b)

        def super_body(s, carry):
            ci0 = s * NBUF
            for b in range(NBUF):
                ci = ci0 + b
                fire(ci + NBUF - 1, (b + NBUF - 1) % NBUF)
                wait(b)
                compute(ci, b)
            return carry

        lax.fori_loop(0, n_super, super_body, 0)
        for ci in range(n_super * NBUF, n_chunks):
            b = ci % NBUF
            if ci + NBUF - 1 < n_chunks:
                fire(ci + NBUF - 1, (b + NBUF - 1) % NBUF)
            wait(b)
            compute(ci, b)
        pltpu.sync_copy(outv, out_hbm.at[pl.ds(base, per_w)])

    return k(x, src, dst)


def kernel(x, edge_index):
    ei = edge_index.astype(jnp.int32)
    scores = _sc_scores(x, ei[0], ei[1])
    return scores.reshape(-1, 1)
